# Initial kernel scaffold; baseline (speedup 1.0000x reference)
import jax
import jax.numpy as jnp
from jax.experimental import pallas as pl


def kernel(x, embedding):
    raise NotImplementedError("write your pallas kernel here")


# speedup vs baseline: 1.6847x; 1.6847x over previous
import functools

import jax
import jax.numpy as jnp
from jax import lax
from jax.experimental import pallas as pl
from jax.experimental.pallas import tpu as pltpu
from jax.experimental.pallas import tpu_sc as plsc

VOCAB = 1000000
DIM = 64
NC = 2
NS = 16
NW = NC * NS

CHUNK = 128


def _build_gather(num_idx: int):
    assert num_idx % (NW * CHUNK) == 0
    b_per_w = num_idx // NW
    nchunk = b_per_w // CHUNK

    mesh = plsc.VectorSubcoreMesh(core_axis_name="c", subcore_axis_name="s")

    @functools.partial(
        pl.kernel,
        mesh=mesh,
        out_type=jax.ShapeDtypeStruct((num_idx, DIM), jnp.float32),
        compiler_params=pltpu.CompilerParams(use_tc_tiling_on_sc=False),
        scratch_types=[
            pltpu.VMEM((nchunk, CHUNK), jnp.int32),
            pltpu.VMEM((CHUNK, DIM), jnp.float32),
            pltpu.SemaphoreType.DMA,
        ],
    )
    def gather_kernel(idx_hbm, table_hbm, out_hbm, idx_v, buf, sem):
        c = lax.axis_index("c")
        s = lax.axis_index("s")
        wid = s * NC + c
        base = wid * b_per_w
        pltpu.sync_copy(idx_hbm.at[wid], idx_v)

        @pl.loop(0, nchunk)
        def _(j):
            pltpu.async_copy(table_hbm.at[idx_v.at[j]], buf, sem).wait()
            pltpu.sync_copy(buf, out_hbm.at[pl.ds(base + j * CHUNK, CHUNK)])

    return gather_kernel


def kernel(x, embedding):
    b, l = x.shape
    num_idx = b * l
    idx = x.reshape(NW, num_idx // (NW * CHUNK), CHUNK).astype(jnp.int32)
    out = _build_gather(num_idx)(idx, embedding)
    return out.reshape(b, l, DIM)


# speedup vs baseline: 1.8767x; 1.1139x over previous
import functools

import jax
import jax.numpy as jnp
from jax import lax
from jax.experimental import pallas as pl
from jax.experimental.pallas import tpu as pltpu
from jax.experimental.pallas import tpu_sc as plsc

VOCAB = 1000000
DIM = 64
NC = 2
NS = 16
NW = NC * NS

CHUNK = 128
G = 4


def _build_gather(num_idx: int):
    assert num_idx % (NW * CHUNK * G * 2) == 0
    b_per_w = num_idx // NW
    nchunk = b_per_w // CHUNK
    ngroups = nchunk // G

    mesh = plsc.VectorSubcoreMesh(core_axis_name="c", subcore_axis_name="s")

    @functools.partial(
        pl.kernel,
        mesh=mesh,
        out_type=jax.ShapeDtypeStruct((num_idx, DIM), jnp.float32),
        compiler_params=pltpu.CompilerParams(use_tc_tiling_on_sc=False),
        scratch_types=[
            pltpu.VMEM((nchunk, CHUNK), jnp.int32),
            pltpu.VMEM((2, G, CHUNK, DIM), jnp.float32),
            pltpu.SemaphoreType.DMA((2,)),
            pltpu.SemaphoreType.DMA((2,)),
        ],
    )
    def gather_kernel(idx_hbm, table_hbm, out_hbm, idx_v, bufs, gsem, wsem):
        c = lax.axis_index("c")
        s = lax.axis_index("s")
        wid = s * NC + c
        base = wid * b_per_w
        pltpu.sync_copy(idx_hbm.at[wid], idx_v)

        for b in range(G):
            pltpu.async_copy(table_hbm.at[idx_v.at[b]], bufs.at[0, b],
                             gsem.at[0])

        @pl.loop(0, ngroups)
        def _(g):
            cur = lax.rem(g, 2)
            nxt = 1 - cur

            @pl.when(g + 1 < ngroups)
            def _():
                @pl.when(g >= 1)
                def _():
                    for b in range(G):
                        pltpu.make_async_copy(
                            bufs.at[nxt, b],
                            out_hbm.at[pl.ds(base, CHUNK)],
                            wsem.at[nxt]).wait()
                for b in range(G):
                    pltpu.async_copy(
                        table_hbm.at[idx_v.at[(g + 1) * G + b]],
                        bufs.at[nxt, b], gsem.at[nxt])

            for b in range(G):
                pltpu.make_async_copy(
                    out_hbm.at[pl.ds(base, CHUNK)],
                    bufs.at[cur, b], gsem.at[cur]).wait()
            for b in range(G):
                pltpu.async_copy(
                    bufs.at[cur, b],
                    out_hbm.at[pl.ds(base + (g * G + b) * CHUNK, CHUNK)],
                    wsem.at[cur])

        for st in range(2):
            for b in range(G):
                pltpu.make_async_copy(
                    bufs.at[st, b],
                    out_hbm.at[pl.ds(base, CHUNK)],
                    wsem.at[st]).wait()

    return gather_kernel


def kernel(x, embedding):
    b, l = x.shape
    num_idx = b * l
    idx = x.reshape(NW, num_idx // (NW * CHUNK), CHUNK).astype(jnp.int32)
    out = _build_gather(num_idx)(idx, embedding)
    return out.reshape(b, l, DIM)
